# baseline (device time: 170985 ns/iter reference)
import jax
import jax.numpy as jnp
from jax import lax
from jax.experimental import pallas as pl
from jax.experimental.pallas import tpu as pltpu

N_DEV = 16
N_LOC = 2
CAP = 56
SLAB = N_LOC * CAP
G = 4
W = 6


def _moe_body(send_ref, w1_ref, w2_ref, ret_ref, recv_ref,
              send_sems, recv_sems, send_sems2, recv_sems2):
    me = lax.axis_index("i")

    bsem = pltpu.get_barrier_semaphore()
    for k in range(N_DEV):
        pl.when(me != k)(
            lambda k=k: pl.semaphore_signal(
                bsem, inc=1, device_id=(k,),
                device_id_type=pl.DeviceIdType.MESH))
    pl.semaphore_wait(bsem, N_DEV - 1)

    d_model = send_ref.shape[-1]

    def disp(r):
        return pltpu.make_async_remote_copy(
            src_ref=send_ref.at[r],
            dst_ref=recv_ref.at[r],
            send_sem=send_sems.at[r],
            recv_sem=recv_sems.at[r],
            device_id=(lax.rem(me + r, N_DEV),),
            device_id_type=pl.DeviceIdType.MESH,
        )

    def comb(r):
        return pltpu.make_async_remote_copy(
            src_ref=recv_ref.at[r],
            dst_ref=ret_ref.at[r],
            send_sem=send_sems2.at[r],
            recv_sem=recv_sems2.at[r],
            device_id=(lax.rem(me + N_DEV - r, N_DEV),),
            device_id_type=pl.DeviceIdType.MESH,
        )

    recv_ref[0] = send_ref[0]
    dispatch = {r: disp(r) for r in range(1, N_DEV)}
    combine = {r: comb(r) for r in range(1, N_DEV)}

    for r in range(1, min(W, N_DEV - 1) + 1):
        dispatch[r].start()

    n_tiles = N_DEV // G
    for j in range(n_tiles):
        for r in range(j * G, (j + 1) * G):
            if r >= 1 and r + W < N_DEV:
                dispatch[r + W].start()
            if r >= 1:
                dispatch[r].wait_recv()

        lo = j * G
        for le in range(N_LOC):
            a = recv_ref[lo:lo + G, le].reshape(G * CAP, d_model)
            h = jnp.maximum(
                jnp.dot(a, w1_ref[le], preferred_element_type=jnp.float32),
                0.0)
            res = jnp.dot(h.astype(jnp.bfloat16), w2_ref[le],
                          preferred_element_type=jnp.float32)
            recv_ref[lo:lo + G, le] = res.astype(jnp.bfloat16).reshape(
                G, CAP, d_model)

        for r in range(lo, lo + G):
            if r == 0:
                ret_ref[0] = recv_ref[0]
            else:
                combine[r].start()

    for r in range(1, N_DEV):
        combine[r].wait_recv()
        combine[r].wait_send()
        dispatch[r].wait_send()


def kernel(x, assign, W1, W2):
    t_per, d_model = x.shape
    me = lax.axis_index("i")

    a = assign.astype(jnp.int32)
    tok = jnp.arange(t_per, dtype=jnp.int32)
    onehot = (a[:, None] == jnp.arange(N_DEV * N_LOC)[None, :]).astype(
        jnp.int32)
    csum = jnp.cumsum(onehot, axis=0) - onehot
    rank = jnp.take_along_axis(csum, a[:, None], axis=1)[:, 0]
    owner = a // N_LOC
    rnd = jnp.remainder(owner - me, N_DEV)
    slot = rnd * SLAB + (a % N_LOC) * CAP + rank

    token_of_slot = jnp.zeros((N_DEV * SLAB,), jnp.int32).at[slot].set(tok)
    send = x[token_of_slot].astype(jnp.bfloat16).reshape(
        N_DEV, N_LOC, CAP, d_model)

    ret = pl.pallas_call(
        _moe_body,
        out_shape=jax.ShapeDtypeStruct((N_DEV, N_LOC, CAP, d_model),
                                       jnp.bfloat16),
        in_specs=[
            pl.BlockSpec(memory_space=pltpu.VMEM),
            pl.BlockSpec(memory_space=pltpu.VMEM),
            pl.BlockSpec(memory_space=pltpu.VMEM),
        ],
        out_specs=pl.BlockSpec(memory_space=pltpu.VMEM),
        scratch_shapes=[
            pltpu.VMEM((N_DEV, N_LOC, CAP, d_model), jnp.bfloat16),
            pltpu.SemaphoreType.DMA((N_DEV,)),
            pltpu.SemaphoreType.DMA((N_DEV,)),
            pltpu.SemaphoreType.DMA((N_DEV,)),
            pltpu.SemaphoreType.DMA((N_DEV,)),
        ],
        compiler_params=pltpu.CompilerParams(
            collective_id=0, vmem_limit_bytes=120 * 1024 * 1024),
    )(send, W1.astype(jnp.bfloat16), W2.astype(jnp.bfloat16))

    return ret.reshape(N_DEV * SLAB, d_model)[slot].astype(jnp.float32)
